# baseline (speedup 1.0000x reference)
import dataclasses
import functools

import jax
import jax.numpy as jnp
import numpy as np
from jax.experimental import pallas as pl
from jax.experimental.pallas import tpu as pltpu
from jax.experimental.pallas import tpu_sc as plsc

B = 1024
D = 64
M = 100000
K = 128
ALPHA = 0.1
BETA = 1e-8
EPS = 1e-3

MT = 2048
NMT = 49
MP = MT * NMT
RT = 256
RG = 8
NEG = -1e30
L = 16
NB = MP // L
CAP = 192
PAD_BIN = NB - 1

NC = 2
NS = 16
SL = 16
ROWS_PER_W = B // (NC * NS)

_INT_MIN = np.int32(-2**31)
_FLIP = np.int32(0x7FFFFFFF)


def _monotone_i32(x):
    b = jax.lax.bitcast_convert_type(x, jnp.int32)
    return jnp.where(b >= 0, b, b ^ _FLIP)


def _inv_monotone_i32(k):
    b = jnp.where(k >= 0, k, k ^ _FLIP)
    return jax.lax.bitcast_convert_type(b, jnp.float32)


def _score_kernel(q_ref, kt_ref, lp_ref, a_ref, u_ref):
    j = pl.program_id(1)
    sim = jnp.dot(q_ref[:], kt_ref[:], preferred_element_type=jnp.float32)
    col = j * MT + jax.lax.broadcasted_iota(jnp.int32, (RT, MT), 1)
    a = jnp.where(col < M, sim + lp_ref[:], NEG)
    a_ref[:] = a
    u_ref[:] = jnp.max(a.reshape(RT, MT // L, L), axis=2)


def _binsel_kernel(u_ref, s_ref, key_ref):
    nch = 7
    cw = NB // nch
    for j in range(nch):
        sl = pl.ds(j * cw, cw)
        key_ref[:, sl] = _monotone_i32(u_ref[:, sl])

    def bit_step(i, t_u):
        bit = jax.lax.shift_left(jnp.int32(1), jnp.int32(31) - i)
        cand_u = t_u | bit
        cand_s = cand_u ^ _INT_MIN
        cnt = jnp.zeros((RG, 1), jnp.int32)
        for j in range(nch):
            k = key_ref[:, pl.ds(j * cw, cw)]
            cnt = cnt + jnp.sum((k >= cand_s).astype(jnp.int32), axis=1,
                                keepdims=True)
        return jnp.where(cnt >= jnp.int32(K), cand_u, t_u)

    t_u = jax.lax.fori_loop(0, 32, bit_step, jnp.zeros((RG, 1), jnp.int32))
    s_ref[:] = _inv_monotone_i32(t_u ^ _INT_MIN)


def _sc_body(u_hbm, srep_hbm, atbl_hbm, gtbl_hbm, gvtbl_hbm,
             ca_hbm, cg_hbm, cgv_hbm,
             u_v, s_v, ilo_v, ihi_v, glo_v, ghi_v, ca_v, cg_v, cgv_v, sem):
    wid = jax.lax.axis_index("s") * NC + jax.lax.axis_index("c")

    @pl.loop(0, ROWS_PER_W)
    def _row(r):
        b = wid * ROWS_PER_W + r
        pltpu.sync_copy(u_hbm.at[b], u_v)
        pltpu.sync_copy(srep_hbm.at[b], s_v)
        pad = jnp.full((SL,), PAD_BIN, jnp.int32)
        for jj in range(CAP // 2 // SL):
            ilo_v[pl.ds(jj * SL, SL)] = pad
            ihi_v[pl.ds(jj * SL, SL)] = pad
        sv = s_v[...]
        half = jnp.int32(CAP // 2)

        def body(j, off):
            uv = u_v[pl.ds(j * SL, SL)]
            m = uv >= sv
            mi = m.astype(jnp.int32)
            c = plsc.cumsum(mi)
            pos = off + c - 1
            src = jax.lax.iota(jnp.int32, SL) + j * SL
            m_lo = jnp.logical_and(m, pos < half)
            m_hi = jnp.logical_and(m, jnp.logical_and(pos >= half,
                                                      pos < jnp.int32(CAP)))
            plsc.store_scatter(ilo_v, [pos], src, mask=m_lo)
            plsc.store_scatter(ihi_v, [pos - half], src, mask=m_hi)
            return off + jnp.sum(mi)

        jax.lax.fori_loop(0, NB // SL, body, jnp.int32(0))

        boff = b * NB
        for jj in range(CAP // 2 // SL):
            sl = pl.ds(jj * SL, SL)
            glo_v[sl] = ilo_v[sl] + boff
            ghi_v[sl] = ihi_v[sl] + boff

        h = CAP // 2
        c1 = pltpu.async_copy(atbl_hbm.at[glo_v], ca_v.at[pl.ds(0, h)], sem)
        c2 = pltpu.async_copy(atbl_hbm.at[ghi_v], ca_v.at[pl.ds(h, h)], sem)
        c3 = pltpu.async_copy(gtbl_hbm.at[ilo_v], cg_v.at[pl.ds(0, h)], sem)
        c4 = pltpu.async_copy(gtbl_hbm.at[ihi_v], cg_v.at[pl.ds(h, h)], sem)
        c5 = pltpu.async_copy(gvtbl_hbm.at[ilo_v], cgv_v.at[pl.ds(0, h)], sem)
        c6 = pltpu.async_copy(gvtbl_hbm.at[ihi_v], cgv_v.at[pl.ds(h, h)], sem)
        c1.wait(); c2.wait(); c3.wait(); c4.wait(); c5.wait(); c6.wait()

        pltpu.sync_copy(ca_v, ca_hbm.at[b])
        pltpu.sync_copy(cg_v, cg_hbm.at[b])
        pltpu.sync_copy(cgv_v, cgv_hbm.at[b])


def _final_kernel(a_ref, g_ref, gv_ref, out_ref, key_ref):
    CE = CAP * L
    nch = 3
    cw = CE // nch
    for j in range(nch):
        sl = pl.ds(j * cw, cw)
        key_ref[:, sl] = _monotone_i32(a_ref[:, sl])

    def bit_step(i, t_u):
        bit = jax.lax.shift_left(jnp.int32(1), jnp.int32(31) - i)
        cand_u = t_u | bit
        cand_s = cand_u ^ _INT_MIN
        cnt = jnp.zeros((RG, 1), jnp.int32)
        for j in range(nch):
            k = key_ref[:, pl.ds(j * cw, cw)]
            cnt = cnt + jnp.sum((k >= cand_s).astype(jnp.int32), axis=1,
                                keepdims=True)
        return jnp.where(cnt >= jnp.int32(K), cand_u, t_u)

    t_u = jax.lax.fori_loop(0, 32, bit_step, jnp.zeros((RG, 1), jnp.int32))
    t_s = t_u ^ _INT_MIN

    s_gt_w = jnp.zeros((RG, 1), jnp.float32)
    s_gt_wv = jnp.zeros((RG, 1), jnp.float32)
    s_eq_w = jnp.zeros((RG, 1), jnp.float32)
    s_eq_wv = jnp.zeros((RG, 1), jnp.float32)
    cnt_gt = jnp.zeros((RG, 1), jnp.float32)
    cnt_eq = jnp.zeros((RG, 1), jnp.float32)
    for j in range(nch):
        sl = pl.ds(j * cw, cw)
        k = key_ref[:, sl]
        e = jnp.exp(a_ref[:, sl] - 1.0)
        w = e * g_ref[:, sl]
        wv = e * gv_ref[:, sl]
        gt = (k > t_s).astype(jnp.float32)
        eq = (k == t_s).astype(jnp.float32)
        s_gt_w += jnp.sum(w * gt, axis=1, keepdims=True)
        s_gt_wv += jnp.sum(wv * gt, axis=1, keepdims=True)
        s_eq_w += jnp.sum(w * eq, axis=1, keepdims=True)
        s_eq_wv += jnp.sum(wv * eq, axis=1, keepdims=True)
        cnt_gt += jnp.sum(gt, axis=1, keepdims=True)
        cnt_eq += jnp.sum(eq, axis=1, keepdims=True)

    frac = (jnp.float32(K) - cnt_gt) / jnp.maximum(cnt_eq, 1.0)
    denom = s_gt_w + frac * s_eq_w
    numer = s_gt_wv + frac * s_eq_wv
    out_ref[:] = jnp.clip(numer / denom, EPS, 1.0 - EPS)


@jax.jit
def kernel(q, memory_key, memory_values, memory_hist):
    kt = jnp.pad(memory_key, ((0, MP - M), (0, 0))).T
    hp = memory_hist + BETA
    lp = jnp.pad(jnp.log(hp), (0, MP - M)).reshape(1, MP)
    g = (ALPHA * memory_hist + BETA) / hp
    gtbl = jnp.pad(g, (0, MP - M)).reshape(NB, L)
    gvtbl = jnp.pad(g * memory_values, (0, MP - M)).reshape(NB, L)

    a, u = pl.pallas_call(
        _score_kernel,
        grid=(B // RT, NMT),
        in_specs=[
            pl.BlockSpec((RT, D), lambda i, j: (i, 0)),
            pl.BlockSpec((D, MT), lambda i, j: (0, j)),
            pl.BlockSpec((1, MT), lambda i, j: (0, j)),
        ],
        out_specs=[
            pl.BlockSpec((RT, MT), lambda i, j: (i, j)),
            pl.BlockSpec((RT, MT // L), lambda i, j: (i, j)),
        ],
        out_shape=[
            jax.ShapeDtypeStruct((B, MP), jnp.float32),
            jax.ShapeDtypeStruct((B, NB), jnp.float32),
        ],
        compiler_params=pltpu.CompilerParams(
            dimension_semantics=("parallel", "arbitrary"),
        ),
    )(q, kt, lp)

    s = pl.pallas_call(
        _binsel_kernel,
        grid=(B // RG,),
        in_specs=[pl.BlockSpec((RG, NB), lambda i: (i, 0))],
        out_specs=pl.BlockSpec((RG, 1), lambda i: (i, 0)),
        out_shape=jax.ShapeDtypeStruct((B, 1), jnp.float32),
        scratch_shapes=[pltpu.VMEM((RG, NB), jnp.int32)],
        compiler_params=pltpu.CompilerParams(
            dimension_semantics=("parallel",),
        ),
    )(u)

    srep = jnp.broadcast_to(s, (B, SL))
    atbl = a.reshape(B * NB, L)

    sc_out_type = [
        jax.ShapeDtypeStruct((B, CAP, L), jnp.float32),
        jax.ShapeDtypeStruct((B, CAP, L), jnp.float32),
        jax.ShapeDtypeStruct((B, CAP, L), jnp.float32),
    ]
    mesh = plsc.VectorSubcoreMesh(core_axis_name="c", subcore_axis_name="s",
                                  num_cores=NC, num_subcores=NS)
    sc_cp = pltpu.CompilerParams(needs_layout_passes=False,
                                 use_tc_tiling_on_sc=False)
    ca, cg, cgv = pl.kernel(
        _sc_body,
        out_type=sc_out_type,
        mesh=mesh,
        compiler_params=sc_cp,
        scratch_types=[
            pltpu.VMEM((NB,), jnp.float32),
            pltpu.VMEM((SL,), jnp.float32),
            pltpu.VMEM((CAP // 2,), jnp.int32),
            pltpu.VMEM((CAP // 2,), jnp.int32),
            pltpu.VMEM((CAP // 2,), jnp.int32),
            pltpu.VMEM((CAP // 2,), jnp.int32),
            pltpu.VMEM((CAP, L), jnp.float32),
            pltpu.VMEM((CAP, L), jnp.float32),
            pltpu.VMEM((CAP, L), jnp.float32),
            pltpu.SemaphoreType.DMA,
        ],
    )(u, srep, atbl, gtbl, gvtbl)

    res = pl.pallas_call(
        _final_kernel,
        grid=(B // RG,),
        in_specs=[
            pl.BlockSpec((RG, CAP * L), lambda i: (i, 0)),
            pl.BlockSpec((RG, CAP * L), lambda i: (i, 0)),
            pl.BlockSpec((RG, CAP * L), lambda i: (i, 0)),
        ],
        out_specs=pl.BlockSpec((RG, 1), lambda i: (i, 0)),
        out_shape=jax.ShapeDtypeStruct((B, 1), jnp.float32),
        scratch_shapes=[pltpu.VMEM((RG, CAP * L), jnp.int32)],
        compiler_params=pltpu.CompilerParams(
            dimension_semantics=("parallel",),
        ),
    )(ca.reshape(B, CAP * L), cg.reshape(B, CAP * L),
      cgv.reshape(B, CAP * L))

    return res.reshape(B)


# speedup vs baseline: 1.4047x; 1.4047x over previous
import functools

import jax
import jax.numpy as jnp
import numpy as np
from jax.experimental import pallas as pl
from jax.experimental.pallas import tpu as pltpu

B = 1024
D = 64
M = 100000
K = 128
ALPHA = 0.1
BETA = 1e-8
EPS = 1e-3

MT = 2048
NMT = 49
MP = MT * NMT
RT = 256
RG = 8
NEG = -1e30

_INT_MIN = np.int32(-2**31)


def _score_kernel(q_ref, kt_ref, out_ref):
    j = pl.program_id(1)
    sim = jnp.dot(q_ref[:], kt_ref[:], preferred_element_type=jnp.float32)
    col = j * MT + jax.lax.broadcasted_iota(jnp.int32, (RT, MT), 1)
    out_ref[:] = jnp.where(col < M, sim, NEG)


def _monotone_i32(x):
    b = jax.lax.bitcast_convert_type(x, jnp.int32)
    return jnp.where(b >= 0, b, b ^ jnp.int32(0x7FFFFFFF))


def _select_kernel(sim_ref, lp_ref, c2_ref, val_ref, out_ref, key_ref):
    nchunks = MP // MT

    for j in range(nchunks):
        sl = pl.ds(j * MT, MT)
        a = sim_ref[:, sl] + lp_ref[:, sl]
        key_ref[:, sl] = _monotone_i32(a)

    kcount = jnp.int32(K)

    def bit_step(i, t_u):
        bit = jax.lax.shift_left(jnp.int32(1), jnp.int32(31) - i)
        cand_u = t_u | bit
        cand_s = cand_u ^ _INT_MIN
        cnt = jnp.zeros((RG, 1), jnp.int32)
        for j in range(nchunks):
            k = key_ref[:, pl.ds(j * MT, MT)]
            m = (k >= cand_s).astype(jnp.int32)
            cnt = cnt + jnp.sum(m, axis=1, keepdims=True)
        return jnp.where(cnt >= kcount, cand_u, t_u)

    t_u = jax.lax.fori_loop(0, 32, bit_step, jnp.zeros((RG, 1), jnp.int32))
    t_s = t_u ^ _INT_MIN

    s_gt_w = jnp.zeros((RG, 1), jnp.float32)
    s_gt_wv = jnp.zeros((RG, 1), jnp.float32)
    s_eq_w = jnp.zeros((RG, 1), jnp.float32)
    s_eq_wv = jnp.zeros((RG, 1), jnp.float32)
    cnt_gt = jnp.zeros((RG, 1), jnp.float32)
    cnt_eq = jnp.zeros((RG, 1), jnp.float32)
    for j in range(nchunks):
        sl = pl.ds(j * MT, MT)
        k = key_ref[:, sl]
        sim = sim_ref[:, sl]
        w = jnp.exp(sim - 1.0) * c2_ref[:, sl]
        wv = w * val_ref[:, sl]
        gt = (k > t_s).astype(jnp.float32)
        eq = (k == t_s).astype(jnp.float32)
        s_gt_w += jnp.sum(w * gt, axis=1, keepdims=True)
        s_gt_wv += jnp.sum(wv * gt, axis=1, keepdims=True)
        s_eq_w += jnp.sum(w * eq, axis=1, keepdims=True)
        s_eq_wv += jnp.sum(wv * eq, axis=1, keepdims=True)
        cnt_gt += jnp.sum(gt, axis=1, keepdims=True)
        cnt_eq += jnp.sum(eq, axis=1, keepdims=True)

    frac = (jnp.float32(K) - cnt_gt) / jnp.maximum(cnt_eq, 1.0)
    denom = s_gt_w + frac * s_eq_w
    numer = s_gt_wv + frac * s_eq_wv
    out_ref[:] = jnp.clip(numer / denom, EPS, 1.0 - EPS)


@jax.jit
def kernel(q, memory_key, memory_values, memory_hist):
    kt = jnp.pad(memory_key, ((0, MP - M), (0, 0))).T
    lp = jnp.pad(jnp.log(memory_hist + BETA), (0, MP - M)).reshape(1, MP)
    c2 = jnp.pad(ALPHA * memory_hist + BETA, (0, MP - M)).reshape(1, MP)
    val = jnp.pad(memory_values, (0, MP - M)).reshape(1, MP)

    sim = pl.pallas_call(
        _score_kernel,
        grid=(B // RT, NMT),
        in_specs=[
            pl.BlockSpec((RT, D), lambda i, j: (i, 0)),
            pl.BlockSpec((D, MT), lambda i, j: (0, j)),
        ],
        out_specs=pl.BlockSpec((RT, MT), lambda i, j: (i, j)),
        out_shape=jax.ShapeDtypeStruct((B, MP), jnp.float32),
        compiler_params=pltpu.CompilerParams(
            dimension_semantics=("parallel", "arbitrary"),
        ),
    )(q, kt)

    res = pl.pallas_call(
        _select_kernel,
        grid=(B // RG,),
        in_specs=[
            pl.BlockSpec((RG, MP), lambda i: (i, 0)),
            pl.BlockSpec((1, MP), lambda i: (0, 0)),
            pl.BlockSpec((1, MP), lambda i: (0, 0)),
            pl.BlockSpec((1, MP), lambda i: (0, 0)),
        ],
        out_specs=pl.BlockSpec((RG, 1), lambda i: (i, 0)),
        out_shape=jax.ShapeDtypeStruct((B, 1), jnp.float32),
        scratch_shapes=[pltpu.VMEM((RG, MP), jnp.int32)],
        compiler_params=pltpu.CompilerParams(
            dimension_semantics=("parallel",),
        ),
    )(sim, lp, c2, val)

    return res.reshape(B)
